# baseline (speedup 1.0000x reference)
import functools

import jax
import jax.numpy as jnp
from jax import lax
from jax.experimental import pallas as pl
from jax.experimental.pallas import tpu as pltpu
from jax.experimental.pallas import tpu_sc as plsc

ROWS = 16384
COLS = 26
DIM = 32
N = ROWS * COLS
TROWS = 250000
TDIM = 128

_INFO = plsc.get_sparse_core_info()
NC = _INFO.num_cores
NS = _INFO.num_subcores
NW = NC * NS
L = 16

PER_W = N // NW
CHUNK = 128
NCHUNK = PER_W // CHUNK
NBUF = 3

_mesh = plsc.VectorSubcoreMesh(core_axis_name="c", subcore_axis_name="s")


@functools.partial(
    pl.kernel,
    mesh=_mesh,
    out_type=jax.ShapeDtypeStruct((N, DIM), jnp.float32),
    compiler_params=pltpu.CompilerParams(needs_layout_passes=False),
    scratch_types=[
        pltpu.VMEM((NBUF, CHUNK), jnp.int32),
        pltpu.VMEM((NBUF, CHUNK), jnp.int32),
        pltpu.VMEM((NBUF, CHUNK, TDIM), jnp.float32),
        pltpu.VMEM((NBUF, CHUNK, DIM), jnp.float32),
        pltpu.SemaphoreType.DMA((NBUF,)),
        pltpu.SemaphoreType.DMA((NBUF,)),
        pltpu.SemaphoreType.DMA((NBUF,)),
    ],
)
def _emb_lookup(idx_hbm, table_hbm, out_hbm, idx_v, gidx_v, tiles_v, rows_v,
                sem_i, sem_g, sem_w):
    wid = lax.axis_index("s") * NC + lax.axis_index("c")
    base = wid * PER_W

    def fire_idx(c, b):
        pltpu.async_copy(idx_hbm.at[wid, c], idx_v.at[b], sem_i.at[b])

    def fire_gathers(c, b):
        pltpu.make_async_copy(
            idx_hbm.at[wid, 0], idx_v.at[b], sem_i.at[b]
        ).wait()
        for j in range(CHUNK // L):
            v = idx_v[b, pl.ds(j * L, L)]
            gidx_v[b, pl.ds(j * L, L)] = jnp.right_shift(v, 2)
        pltpu.async_copy(
            table_hbm.at[gidx_v.at[b]], tiles_v.at[b], sem_g.at[b]
        )

    def wait_gathers(b):
        pltpu.make_async_copy(
            table_hbm.at[pl.ds(0, CHUNK)], tiles_v.at[b], sem_g.at[b]
        ).wait()

    def fire_writeback(c, b):
        pltpu.async_copy(
            rows_v.at[b], out_hbm.at[pl.ds(base + c * CHUNK, CHUNK)], sem_w.at[b]
        )

    def wait_writeback(b):
        pltpu.make_async_copy(
            rows_v.at[b], out_hbm.at[pl.ds(0, CHUNK)], sem_w.at[b]
        ).wait()

    def extract(c, b):
        iota = lax.iota(jnp.int32, L)

        def rg_body(rg, _):
            r0 = rg * L
            vi = idx_v[b, pl.ds(r0, L)]
            colbase = jnp.left_shift(jnp.bitwise_and(vi, 3), 5)
            rowvec = iota + r0
            for d in range(DIM):
                vals = plsc.load_gather(
                    tiles_v.at[b], [rowvec, colbase + d]
                )
                plsc.store_scatter(
                    rows_v.at[b],
                    [rowvec, jnp.full((L,), d, jnp.int32)],
                    vals,
                )
            return ()

        lax.fori_loop(0, CHUNK // L, rg_body, (), unroll=False)

    fire_idx(0, 0)
    fire_idx(1, 1)
    fire_gathers(0, 0)
    fire_idx(2, 2)
    fire_gathers(1, 1)

    def chunk_body(c, _):
        for b in range(NBUF):
            cc = c * NBUF + b
            wait_gathers(b)

            @pl.when(cc >= NBUF)
            def _():
                wait_writeback(b)

            extract(cc, b)
            fire_writeback(cc, b)

            @pl.when(cc + NBUF < NCHUNK)
            def _():
                fire_idx(cc + NBUF, b)

            @pl.when(cc + 2 < NCHUNK)
            def _():
                fire_gathers(cc + 2, (b + 2) % NBUF)

        return ()

    lax.fori_loop(0, NCHUNK // NBUF, chunk_body, (), unroll=False)

    for cc, b in ((NCHUNK - 2, (NCHUNK - 2) % NBUF), (NCHUNK - 1, (NCHUNK - 1) % NBUF)):
        wait_gathers(b)
        wait_writeback(b)
        extract(cc, b)
        fire_writeback(cc, b)

    for b in range(NBUF):
        wait_writeback(b)


def kernel(x, weight):
    idx = x.reshape(NW, NCHUNK, CHUNK)
    table = weight.reshape(TROWS, TDIM)
    out = _emb_lookup(idx, table)
    return out.reshape(ROWS, COLS, DIM)
